# baseline (device time: 36027 ns/iter reference)
import jax
import jax.numpy as jnp
from jax import lax
from jax.experimental import pallas as pl
from jax.experimental.pallas import tpu as pltpu

N_DEV = 16
N_TOK = 512
D_OUT = 512
E_PER = 2
CHUNK = N_TOK // N_DEV
N_STEP = 4
N_FLOW = 2
COL_H = D_OUT // N_FLOW

ORDERS = ((0, 2, 1, 3), (2, 0, 3, 1))

RS_ROWS = [(1 << (N_STEP - 1 - t)) * CHUNK for t in range(N_STEP)]
RS_OFF = [sum(RS_ROWS[:t]) for t in range(N_STEP)]
BUF_ROWS = sum(RS_ROWS)


def _virt(d, order):
    v = 0
    for t, k in enumerate(order):
        v = v + ((d >> k) & 1) * (1 << (N_STEP - 1 - t))
    return v


def _real(q, order):
    r = 0
    for vb in range(N_STEP):
        r = r + ((q >> vb) & 1) * (1 << order[N_STEP - 1 - vb])
    return r


_PERMS = [[_virt(c, o) for c in range(N_DEV)] for o in ORDERS]


def kernel(x, router_W, route_idx, expert_W):
    del router_W

    def body(x_ref, idx_ref, w_ref, out_ref, acc_ref, bf_buf,
             rs_send, rs_recv, ag_send, ag_recv):
        my = lax.axis_index("i")
        virt = [_virt(my, o) for o in ORDERS]
        pending_sends = []

        barrier = pltpu.get_barrier_semaphore()
        for b in range(N_STEP):
            pl.semaphore_signal(barrier, inc=1, device_id=(my ^ (1 << b),),
                                device_id_type=pl.DeviceIdType.MESH)
        pl.semaphore_wait(barrier, N_STEP)

        route = idx_ref[:, :]
        x_all = x_ref[:, :]
        e0 = my * E_PER
        xm = [x_all * (route == (e0 + k)).astype(jnp.float32)
              for k in range(E_PER)]

        def rs_start(f, t):
            vb = N_STEP - 1 - t
            send = (((virt[f] >> vb) << vb) ^ (1 << vb)) * CHUNK
            rdma = pltpu.make_async_remote_copy(
                src_ref=acc_ref.at[f, pl.ds(send, RS_ROWS[t]), :],
                dst_ref=bf_buf.at[f, pl.ds(RS_OFF[t], RS_ROWS[t]), :],
                send_sem=rs_send.at[t, f],
                recv_sem=rs_recv.at[t, f],
                device_id=(my ^ (1 << ORDERS[f][t]),),
                device_id_type=pl.DeviceIdType.MESH,
            )
            rdma.start()
            pending_sends.append(rdma)
            return rdma

        def rs_add(f, t):
            vb = N_STEP - 1 - t
            keep = ((virt[f] >> vb) << vb) * CHUNK
            acc_ref[f, pl.ds(keep, RS_ROWS[t]), :] = (
                acc_ref[f, pl.ds(keep, RS_ROWS[t]), :]
                + bf_buf[f, pl.ds(RS_OFF[t], RS_ROWS[t]), :])

        rs_rdmas = [[None] * N_STEP for _ in range(N_FLOW)]
        for f in range(N_FLOW):
            partial = jnp.zeros((N_TOK, COL_H), jnp.float32)
            for k in range(E_PER):
                wk = w_ref[k, :, pl.ds(f * COL_H, COL_H)]
                partial = partial + jnp.dot(
                    xm[k], wk, preferred_element_type=jnp.float32)
            for c in range(N_DEV):
                acc_ref[f, pl.ds(_PERMS[f][c] * CHUNK, CHUNK), :] = lax.slice(
                    partial, (c * CHUNK, 0), ((c + 1) * CHUNK, COL_H))
            rs_rdmas[f][0] = rs_start(f, 0)

        for t in range(1, N_STEP):
            for f in range(N_FLOW):
                rs_rdmas[f][t - 1].wait_recv()
                rs_add(f, t - 1)
                rs_rdmas[f][t] = rs_start(f, t)
        for f in range(N_FLOW):
            rs_rdmas[f][N_STEP - 1].wait_recv()
            rs_add(f, N_STEP - 1)

        def ag_start(f, u):
            send = ((virt[f] >> u) << u) * CHUNK
            rdma = pltpu.make_async_remote_copy(
                src_ref=acc_ref.at[f, pl.ds(send, (1 << u) * CHUNK), :],
                dst_ref=acc_ref.at[f, pl.ds(send, (1 << u) * CHUNK), :],
                send_sem=ag_send.at[u, f],
                recv_sem=ag_recv.at[u, f],
                device_id=(my ^ (1 << ORDERS[f][N_STEP - 1 - u]),),
                device_id_type=pl.DeviceIdType.MESH,
            )
            rdma.start()
            pending_sends.append(rdma)
            return rdma

        def unperm_store(f, q_base, n_chunks):
            for j in range(n_chunks):
                q = q_base + j
                r = _real(q, ORDERS[f])
                out_ref[pl.ds(r * CHUNK, CHUNK),
                        pl.ds(f * COL_H, COL_H)] = acc_ref[
                    f, pl.ds(q * CHUNK, CHUNK), :]

        ag_rdmas = [[None] * N_STEP for _ in range(N_FLOW)]
        for f in range(N_FLOW):
            ag_rdmas[f][0] = ag_start(f, 0)
        for f in range(N_FLOW):
            out_ref[pl.ds(my * CHUNK, CHUNK),
                    pl.ds(f * COL_H, COL_H)] = acc_ref[
                f, pl.ds(virt[f] * CHUNK, CHUNK), :]
        for u in range(1, N_STEP):
            for f in range(N_FLOW):
                ag_rdmas[f][u - 1].wait_recv()
                ag_rdmas[f][u] = ag_start(f, u)
                inc = (((virt[f] >> (u - 1)) << (u - 1)) ^ (1 << (u - 1)))
                unperm_store(f, inc, 1 << (u - 1))
        for f in range(N_FLOW):
            ag_rdmas[f][N_STEP - 1].wait_recv()
            inc = (((virt[f] >> (N_STEP - 1)) << (N_STEP - 1))
                   ^ (1 << (N_STEP - 1)))
            unperm_store(f, inc, 1 << (N_STEP - 1))

        for rdma in pending_sends:
            rdma.wait_send()

    return pl.pallas_call(
        body,
        out_shape=jax.ShapeDtypeStruct((N_TOK, D_OUT), jnp.float32),
        in_specs=[
            pl.BlockSpec(memory_space=pltpu.VMEM),
            pl.BlockSpec(memory_space=pltpu.VMEM),
            pl.BlockSpec(memory_space=pltpu.VMEM),
        ],
        out_specs=pl.BlockSpec(memory_space=pltpu.VMEM),
        scratch_shapes=[
            pltpu.VMEM((N_FLOW, N_TOK, COL_H), jnp.float32),
            pltpu.VMEM((N_FLOW, BUF_ROWS, COL_H), jnp.float32),
            pltpu.SemaphoreType.DMA((N_STEP, N_FLOW)),
            pltpu.SemaphoreType.DMA((N_STEP, N_FLOW)),
            pltpu.SemaphoreType.DMA((N_STEP, N_FLOW)),
            pltpu.SemaphoreType.DMA((N_STEP, N_FLOW)),
        ],
        compiler_params=pltpu.CompilerParams(collective_id=0),
    )(x, route_idx, expert_W)


# device time: 35907 ns/iter; 1.0033x vs baseline; 1.0033x over previous
import jax
import jax.numpy as jnp
from jax import lax
from jax.experimental import pallas as pl
from jax.experimental.pallas import tpu as pltpu

N_DEV = 16
N_TOK = 512
D_OUT = 512
E_PER = 2
CHUNK = N_TOK // N_DEV
N_STEP = 4
N_FLOW = 2
COL_H = D_OUT // N_FLOW

ORDERS = ((0, 2, 1, 3), (2, 0, 3, 1))

RS_ROWS = [(1 << (N_STEP - 1 - t)) * CHUNK for t in range(N_STEP)]
RS_OFF = [sum(RS_ROWS[:t]) for t in range(N_STEP)]
BUF_ROWS = sum(RS_ROWS)


def _virt(d, order):
    v = 0
    for t, k in enumerate(order):
        v = v + ((d >> k) & 1) * (1 << (N_STEP - 1 - t))
    return v


def _real(q, order):
    r = 0
    for vb in range(N_STEP):
        r = r + ((q >> vb) & 1) * (1 << order[N_STEP - 1 - vb])
    return r


_PERMS = [[_virt(c, o) for c in range(N_DEV)] for o in ORDERS]


def kernel(x, router_W, route_idx, expert_W):
    del router_W

    def body(x_ref, idx_ref, w_ref, out_ref, acc_ref, bf_buf,
             rs_send, rs_recv, ag_send, ag_recv):
        my = lax.axis_index("i")
        virt = [_virt(my, o) for o in ORDERS]
        pending_sends = []

        barrier = pltpu.get_barrier_semaphore()
        for b in range(N_STEP):
            pl.semaphore_signal(barrier, inc=1, device_id=(my ^ (1 << b),),
                                device_id_type=pl.DeviceIdType.MESH)
        pl.semaphore_wait(barrier, N_STEP)

        route = idx_ref[:, :]
        x_all = x_ref[:, :]
        e0 = my * E_PER
        xm = [x_all * (route == (e0 + k)).astype(jnp.float32)
              for k in range(E_PER)]

        def rs_start(f, t):
            vb = N_STEP - 1 - t
            send = (((virt[f] >> vb) << vb) ^ (1 << vb)) * CHUNK
            rdma = pltpu.make_async_remote_copy(
                src_ref=acc_ref.at[f, pl.ds(send, RS_ROWS[t]), :],
                dst_ref=bf_buf.at[f, pl.ds(RS_OFF[t], RS_ROWS[t]), :],
                send_sem=rs_send.at[t, f],
                recv_sem=rs_recv.at[t, f],
                device_id=(my ^ (1 << ORDERS[f][t]),),
                device_id_type=pl.DeviceIdType.MESH,
            )
            rdma.start()
            pending_sends.append(rdma)
            return rdma

        def rs_add(f, t):
            vb = N_STEP - 1 - t
            keep = ((virt[f] >> vb) << vb) * CHUNK
            acc_ref[f, pl.ds(keep, RS_ROWS[t]), :] = (
                acc_ref[f, pl.ds(keep, RS_ROWS[t]), :]
                + bf_buf[f, pl.ds(RS_OFF[t], RS_ROWS[t]), :])

        rs_rdmas = [[None] * N_STEP for _ in range(N_FLOW)]
        for f in range(N_FLOW):
            partial = jnp.zeros((N_TOK, COL_H), jnp.float32)
            for k in range(E_PER):
                wk = w_ref[k, :, pl.ds(f * COL_H, COL_H)]
                partial = partial + jnp.dot(
                    xm[k], wk, preferred_element_type=jnp.float32)
            for c in range(N_DEV):
                acc_ref[f, pl.ds(_PERMS[f][c] * CHUNK, CHUNK), :] = lax.slice(
                    partial, (c * CHUNK, 0), ((c + 1) * CHUNK, COL_H))
            rs_rdmas[f][0] = rs_start(f, 0)

        for t in range(1, N_STEP):
            for f in range(N_FLOW):
                rs_rdmas[f][t - 1].wait_recv()
                rs_add(f, t - 1)
                rs_rdmas[f][t] = rs_start(f, t)

        def ag_start(f, u):
            send = ((virt[f] >> u) << u) * CHUNK
            rdma = pltpu.make_async_remote_copy(
                src_ref=acc_ref.at[f, pl.ds(send, (1 << u) * CHUNK), :],
                dst_ref=acc_ref.at[f, pl.ds(send, (1 << u) * CHUNK), :],
                send_sem=ag_send.at[u, f],
                recv_sem=ag_recv.at[u, f],
                device_id=(my ^ (1 << ORDERS[f][N_STEP - 1 - u]),),
                device_id_type=pl.DeviceIdType.MESH,
            )
            rdma.start()
            pending_sends.append(rdma)
            return rdma

        ag_rdmas = [[None] * N_STEP for _ in range(N_FLOW)]
        for f in range(N_FLOW):
            rs_rdmas[f][N_STEP - 1].wait_recv()
            rs_add(f, N_STEP - 1)
            ag_rdmas[f][0] = ag_start(f, 0)
        for u in range(1, N_STEP):
            for f in range(N_FLOW):
                ag_rdmas[f][u - 1].wait_recv()
                ag_rdmas[f][u] = ag_start(f, u)
        for f in range(N_FLOW):
            ag_rdmas[f][N_STEP - 1].wait_recv()
            for c in range(N_DEV):
                out_ref[pl.ds(c * CHUNK, CHUNK),
                        pl.ds(f * COL_H, COL_H)] = acc_ref[
                    f, pl.ds(_PERMS[f][c] * CHUNK, CHUNK), :]

        for rdma in pending_sends:
            rdma.wait_send()

    return pl.pallas_call(
        body,
        out_shape=jax.ShapeDtypeStruct((N_TOK, D_OUT), jnp.float32),
        in_specs=[
            pl.BlockSpec(memory_space=pltpu.VMEM),
            pl.BlockSpec(memory_space=pltpu.VMEM),
            pl.BlockSpec(memory_space=pltpu.VMEM),
        ],
        out_specs=pl.BlockSpec(memory_space=pltpu.VMEM),
        scratch_shapes=[
            pltpu.VMEM((N_FLOW, N_TOK, COL_H), jnp.float32),
            pltpu.VMEM((N_FLOW, BUF_ROWS, COL_H), jnp.float32),
            pltpu.SemaphoreType.DMA((N_STEP, N_FLOW)),
            pltpu.SemaphoreType.DMA((N_STEP, N_FLOW)),
            pltpu.SemaphoreType.DMA((N_STEP, N_FLOW)),
            pltpu.SemaphoreType.DMA((N_STEP, N_FLOW)),
        ],
        compiler_params=pltpu.CompilerParams(collective_id=0),
    )(x, route_idx, expert_W)
